# baseline (speedup 1.0000x reference)
import jax
import jax.numpy as jnp
from jax.experimental import pallas as pl
from jax.experimental.pallas import tpu as pltpu


def _make_prep(n, npad, nh):
    def body(x_ref, W1_ref, W2_ref, b2_ref, Wlin_ref, blin_ref,
             s1_ref, wv_ref, c_ref):
        s1_ref[:n, :] = jnp.dot(x_ref[...], W1_ref[...],
                                preferred_element_type=jnp.float32
                                ).astype(jnp.bfloat16)
        if npad > n:
            s1_ref[n:, :] = jnp.zeros((npad - n, nh), jnp.bfloat16)
        wv_ref[...] = jnp.dot(W2_ref[...], Wlin_ref[...],
                              preferred_element_type=jnp.float32)
        c_ref[...] = jnp.dot(b2_ref[...], Wlin_ref[...],
                             preferred_element_type=jnp.float32) + blin_ref[...]
    return body


def _col_mask(blk, valid):
    lanes = jax.lax.broadcasted_iota(jnp.int32, blk.shape, 1)
    return jnp.where(lanes < valid, blk, 0.0)


def _row_mask(blk, valid):
    rows = jax.lax.broadcasted_iota(jnp.int32, blk.shape, 0)
    return jnp.where(rows < valid, blk, 0.0)


def _widen_v(vb, nh):
    vh = vb.astype(jnp.bfloat16)
    vl = (vb - vh.astype(jnp.float32)).astype(jnp.bfloat16)
    zeros = jnp.zeros((vb.shape[0], nh - 2), jnp.bfloat16)
    return jnp.concatenate([vh, vl, zeros], axis=1)


def _collapse(ow):
    return ow[:, 0:1] + ow[:, 1:2]


def _make_pass1(n, bm, nblk, nh, valid_last):
    last = nblk - 1
    ragged = valid_last < bm

    def body(adj_ref, s1_ref, b1_ref, wv_ref, c_ref, vw_ref, part_ref,
             h_scr, diag_scr, ow_scr):
        ib = pl.program_id(0)
        jb = pl.program_id(1)

        @pl.when(jb == 0)
        def _init():
            h_scr[...] = jnp.broadcast_to(b1_ref[...], h_scr.shape)
            ow_scr[...] = jnp.zeros_like(ow_scr)

        def _step(a16):
            h_scr[...] += jnp.dot(a16, s1_ref[pl.ds(jb * bm, bm), :],
                                  preferred_element_type=jnp.float32)

            @pl.when(jb == ib)
            def _stash_diag():
                diag_scr[...] = a16

            @pl.when(jb < ib)
            def _lower_matvec():
                ow_scr[...] += jnp.dot(a16, vw_ref[pl.ds(jb * bm, bm), :],
                                       preferred_element_type=jnp.float32)

        if ragged:
            @pl.when(jb < last)
            def _main():
                _step(adj_ref[...].astype(jnp.bfloat16))

            @pl.when(jb == last)
            def _edge():
                _step(_col_mask(adj_ref[...], valid_last
                                ).astype(jnp.bfloat16))
        else:
            _step(adj_ref[...].astype(jnp.bfloat16))

        @pl.when(jb == last)
        def _finalize_row():
            hr = jnp.maximum(h_scr[...], 0.0)
            vb = jnp.dot(hr, wv_ref[...],
                         preferred_element_type=jnp.float32)
            vb = _row_mask(vb, n - ib * bm)
            vw = _widen_v(vb, nh)
            vw_ref[pl.ds(ib * bm, bm), :] = vw
            ow_scr[...] += jnp.dot(diag_scr[...], vw,
                                   preferred_element_type=jnp.float32)
            part_ref[...] = _collapse(ow_scr[...]) + c_ref[...]

    return body


def _make_pass2(bm, nblk, valid_last):
    last = nblk - 1
    ragged = valid_last < bm

    def body(adj_ref, vw_ref, part_ref, out_ref, ow_scr):
        ib = pl.program_id(0)
        jb = pl.program_id(1)

        @pl.when(jb == 0)
        def _init():
            ow_scr[...] = jnp.zeros_like(ow_scr)

        def _acc(a16):
            ow_scr[...] += jnp.dot(a16, vw_ref[...],
                                   preferred_element_type=jnp.float32)

        if ragged:
            @pl.when(jnp.logical_and(jb > ib, jb < last))
            def _upper():
                _acc(adj_ref[...].astype(jnp.bfloat16))

            @pl.when(jnp.logical_and(jb > ib, jb == last))
            def _upper_edge():
                _acc(_col_mask(adj_ref[...], valid_last
                               ).astype(jnp.bfloat16))
        else:
            @pl.when(jb > ib)
            def _upper():
                _acc(adj_ref[...].astype(jnp.bfloat16))

        @pl.when(jb == last)
        def _finalize_row():
            out_ref[...] = part_ref[...] + _collapse(ow_scr[...])

    return body


def _pick_bm(n):
    if n >= 8192:
        return 1280
    return max(128, (n // (4 * 128)) * 128) if n >= 512 else 128


def kernel(adj, x, W1, b1, W2, b2, Wlin, blin):
    n, nf = x.shape
    nh = W1.shape[1]
    bm = _pick_bm(n)
    nblk = -(-n // bm)
    npad = nblk * bm
    valid_last = n - (nblk - 1) * bm

    s1, wv, c = pl.pallas_call(
        _make_prep(n, npad, nh),
        out_shape=[
            jax.ShapeDtypeStruct((npad, nh), jnp.bfloat16),
            jax.ShapeDtypeStruct((nh, 1), jnp.float32),
            jax.ShapeDtypeStruct((1, 1), jnp.float32),
        ],
    )(x, W1, W2, b2.reshape(1, nh), Wlin, blin.reshape(1, 1))

    vw, part = pl.pallas_call(
        _make_pass1(n, bm, nblk, nh, valid_last),
        grid=(nblk, nblk),
        in_specs=[
            pl.BlockSpec((bm, bm), lambda i, j: (i, j)),
            pl.BlockSpec((npad, nh), lambda i, j: (0, 0)),
            pl.BlockSpec((1, nh), lambda i, j: (0, 0)),
            pl.BlockSpec((nh, 1), lambda i, j: (0, 0)),
            pl.BlockSpec((1, 1), lambda i, j: (0, 0)),
        ],
        out_specs=[
            pl.BlockSpec((npad, nh), lambda i, j: (0, 0)),
            pl.BlockSpec((bm, 1), lambda i, j: (i, 0)),
        ],
        out_shape=[
            jax.ShapeDtypeStruct((npad, nh), jnp.bfloat16),
            jax.ShapeDtypeStruct((npad, 1), jnp.float32),
        ],
        scratch_shapes=[
            pltpu.VMEM((bm, nh), jnp.float32),
            pltpu.VMEM((bm, bm), jnp.bfloat16),
            pltpu.VMEM((bm, nh), jnp.float32),
        ],
        compiler_params=pltpu.CompilerParams(
            dimension_semantics=("arbitrary", "arbitrary")),
    )(adj, s1, b1.reshape(1, nh), wv, c)

    last = nblk - 1

    def _adj2_idx(i, j):
        return (i, jnp.minimum(jnp.maximum(j, i + 1), last))

    def _vw2_idx(i, j):
        return (jnp.minimum(jnp.maximum(j, i + 1), last), 0)

    out = pl.pallas_call(
        _make_pass2(bm, nblk, valid_last),
        grid=(nblk, nblk),
        in_specs=[
            pl.BlockSpec((bm, bm), _adj2_idx),
            pl.BlockSpec((bm, nh), _vw2_idx),
            pl.BlockSpec((bm, 1), lambda i, j: (i, 0)),
        ],
        out_specs=pl.BlockSpec((bm, 1), lambda i, j: (i, 0)),
        out_shape=jax.ShapeDtypeStruct((npad, 1), jnp.float32),
        scratch_shapes=[
            pltpu.VMEM((bm, nh), jnp.float32),
        ],
        compiler_params=pltpu.CompilerParams(
            dimension_semantics=("arbitrary", "arbitrary")),
    )(adj, vw, part)

    return out[:n]
